# baseline (speedup 1.0000x reference)
import functools

import jax
import jax.numpy as jnp
from jax import lax
from jax.experimental import pallas as pl
from jax.experimental.pallas import tpu as pltpu
from jax.experimental.pallas import tpu_sc as plsc

B = 16384
D = 64
NW = 32
ROWS_PER_W = B // NW
CHUNK = 128
NCHUNK = ROWS_PER_W // CHUNK

H1 = 96
H2 = 64
BK = 2048


def _gather_body(uidx_hbm, midx_hbm, utab_hbm, mtab_hbm, emb_hbm,
                 uidx_v, midx_v, urows_v, mrows_v, gsem, osem):
    wid = lax.axis_index("s") * 2 + lax.axis_index("c")
    base_chunk = wid * NCHUNK
    base_row = wid * ROWS_PER_W
    pltpu.sync_copy(uidx_hbm.at[pl.ds(base_chunk, NCHUNK), :], uidx_v)
    pltpu.sync_copy(midx_hbm.at[pl.ds(base_chunk, NCHUNK), :], midx_v)

    def fire(c):
        s = c % 2
        return (
            pltpu.async_copy(utab_hbm.at[uidx_v.at[c]], urows_v.at[s], gsem),
            pltpu.async_copy(mtab_hbm.at[midx_v.at[c]], mrows_v.at[s], gsem),
        )

    def drain(c, g):
        s = c % 2
        for cp in g:
            cp.wait()
        row0 = base_row + c * CHUNK
        return (
            pltpu.async_copy(
                urows_v.at[s], emb_hbm.at[pl.ds(row0, CHUNK), pl.ds(0, D)],
                osem),
            pltpu.async_copy(
                mrows_v.at[s], emb_hbm.at[pl.ds(row0, CHUNK), pl.ds(D, D)],
                osem),
        )

    outs = []
    g = fire(0)
    for c in range(NCHUNK):
        g_next = fire(c + 1) if c + 1 < NCHUNK else None
        if c >= 2:
            for cp in outs[c - 2]:
                cp.wait()
        outs.append(drain(c, g))
        g = g_next
    for pair in outs[-2:]:
        for cp in pair:
            cp.wait()


@functools.lru_cache(maxsize=None)
def _build_gather_call():
    return pl.kernel(
        _gather_body,
        mesh=plsc.VectorSubcoreMesh(core_axis_name="c", subcore_axis_name="s"),
        out_type=jax.ShapeDtypeStruct((B, 2 * D), jnp.bfloat16),
        scratch_types=[
            pltpu.VMEM((NCHUNK, CHUNK), jnp.int32),
            pltpu.VMEM((NCHUNK, CHUNK), jnp.int32),
            pltpu.VMEM((2, CHUNK, D), jnp.bfloat16),
            pltpu.VMEM((2, CHUNK, D), jnp.bfloat16),
            pltpu.SemaphoreType.DMA,
            pltpu.SemaphoreType.DMA,
        ],
        compiler_params=pltpu.CompilerParams(use_tc_tiling_on_sc=False),
    )


def _mlp_body(emb_ref, w1_ref, b1_ref, w2_ref, b2_ref, w3_ref, b3_ref,
              out_ref):
    e = emb_ref[...].astype(jnp.float32)
    h = jnp.dot(e, w1_ref[...], preferred_element_type=jnp.float32)
    h = jnp.maximum(h + b1_ref[...], 0.0)
    h = jnp.dot(h, w2_ref[...], preferred_element_type=jnp.float32) + b2_ref[...]
    h = jnp.maximum(h, 0.0)
    z = jnp.sum(h * w3_ref[...], axis=1) + b3_ref[0, 0]
    out_ref[...] = jax.nn.sigmoid(z)


_mlp_call = pl.pallas_call(
    _mlp_body,
    grid=(B // BK,),
    in_specs=[
        pl.BlockSpec((BK, 2 * D), lambda i: (i, 0)),
        pl.BlockSpec((2 * D, H1), lambda i: (0, 0)),
        pl.BlockSpec((1, H1), lambda i: (0, 0)),
        pl.BlockSpec((H1, H2), lambda i: (0, 0)),
        pl.BlockSpec((1, H2), lambda i: (0, 0)),
        pl.BlockSpec((1, H2), lambda i: (0, 0)),
        pl.BlockSpec((1, 1), lambda i: (0, 0)),
    ],
    out_specs=pl.BlockSpec((BK,), lambda i: (i,)),
    out_shape=jax.ShapeDtypeStruct((B,), jnp.float32),
)


def kernel(x, user_table, movie_table, W1, b1, W2, b2, W3, b3):
    idx2 = x.astype(jnp.int32) * 2
    uidx = idx2[:, 0].reshape(B // CHUNK, CHUNK)
    midx = idx2[:, 1].reshape(B // CHUNK, CHUNK)
    emb = _build_gather_call()(
        uidx, midx,
        user_table.astype(jnp.bfloat16), movie_table.astype(jnp.bfloat16))
    return _mlp_call(
        emb,
        W1, b1.reshape(1, H1),
        W2, b2.reshape(1, H2),
        W3[:, 0].reshape(1, H2), b3.reshape(1, 1),
    )


# speedup vs baseline: 1.3259x; 1.3259x over previous
import functools

import jax
import jax.numpy as jnp
from jax import lax
from jax.experimental import pallas as pl
from jax.experimental.pallas import tpu as pltpu
from jax.experimental.pallas import tpu_sc as plsc

B = 16384
D = 64
NW = 32
ROWS_PER_W = B // NW
CHUNK = 128
NCHUNK = ROWS_PER_W // CHUNK

H1 = 96
H2 = 64
BK = 2048


def _gather_body(uidx_hbm, midx_hbm, utab_hbm, mtab_hbm, emb_hbm,
                 uidx_v, midx_v, urows_v, mrows_v, gsem, osem):
    wid = lax.axis_index("s") * 2 + lax.axis_index("c")
    base_chunk = wid * NCHUNK
    base_row = wid * ROWS_PER_W
    pltpu.sync_copy(uidx_hbm.at[pl.ds(base_chunk, NCHUNK), :], uidx_v)
    pltpu.sync_copy(midx_hbm.at[pl.ds(base_chunk, NCHUNK), :], midx_v)

    def fire(c):
        s = c % 2
        return (
            pltpu.async_copy(utab_hbm.at[uidx_v.at[c]], urows_v.at[s], gsem),
            pltpu.async_copy(mtab_hbm.at[midx_v.at[c]], mrows_v.at[s], gsem),
        )

    def drain(c, g):
        s = c % 2
        for cp in g:
            cp.wait()
        row0 = base_row + c * CHUNK
        return (
            pltpu.async_copy(
                urows_v.at[s], emb_hbm.at[pl.ds(row0, CHUNK), pl.ds(0, D)],
                osem),
            pltpu.async_copy(
                mrows_v.at[s], emb_hbm.at[pl.ds(row0, CHUNK), pl.ds(D, D)],
                osem),
        )

    outs = []
    g = fire(0)
    for c in range(NCHUNK):
        g_next = fire(c + 1) if c + 1 < NCHUNK else None
        if c >= 2:
            for cp in outs[c - 2]:
                cp.wait()
        outs.append(drain(c, g))
        g = g_next
    for pair in outs[-2:]:
        for cp in pair:
            cp.wait()


@functools.lru_cache(maxsize=None)
def _build_gather_call():
    return pl.kernel(
        _gather_body,
        mesh=plsc.VectorSubcoreMesh(core_axis_name="c", subcore_axis_name="s"),
        out_type=jax.ShapeDtypeStruct((B, 2 * D), jnp.float32),
        scratch_types=[
            pltpu.VMEM((NCHUNK, CHUNK), jnp.int32),
            pltpu.VMEM((NCHUNK, CHUNK), jnp.int32),
            pltpu.VMEM((2, CHUNK, D), jnp.float32),
            pltpu.VMEM((2, CHUNK, D), jnp.float32),
            pltpu.SemaphoreType.DMA,
            pltpu.SemaphoreType.DMA,
        ],
        compiler_params=pltpu.CompilerParams(use_tc_tiling_on_sc=False),
    )


def _mlp_body(emb_ref, w1_ref, b1_ref, w2_ref, b2_ref, w3_ref, b3_ref,
              out_ref):
    e = emb_ref[...]
    h = jnp.dot(e, w1_ref[...], preferred_element_type=jnp.float32)
    h = jnp.maximum(h + b1_ref[...], 0.0)
    h = jnp.dot(h, w2_ref[...], preferred_element_type=jnp.float32) + b2_ref[...]
    h = jnp.maximum(h, 0.0)
    z = jnp.sum(h * w3_ref[...], axis=1) + b3_ref[0, 0]
    out_ref[...] = jax.nn.sigmoid(z)


_mlp_call = pl.pallas_call(
    _mlp_body,
    grid=(B // BK,),
    in_specs=[
        pl.BlockSpec((BK, 2 * D), lambda i: (i, 0)),
        pl.BlockSpec((2 * D, H1), lambda i: (0, 0)),
        pl.BlockSpec((1, H1), lambda i: (0, 0)),
        pl.BlockSpec((H1, H2), lambda i: (0, 0)),
        pl.BlockSpec((1, H2), lambda i: (0, 0)),
        pl.BlockSpec((1, H2), lambda i: (0, 0)),
        pl.BlockSpec((1, 1), lambda i: (0, 0)),
    ],
    out_specs=pl.BlockSpec((BK,), lambda i: (i,)),
    out_shape=jax.ShapeDtypeStruct((B,), jnp.float32),
)


def kernel(x, user_table, movie_table, W1, b1, W2, b2, W3, b3):
    idx2 = x.astype(jnp.int32) * 2
    uidx = idx2[:, 0].reshape(B // CHUNK, CHUNK)
    midx = idx2[:, 1].reshape(B // CHUNK, CHUNK)
    emb = _build_gather_call()(uidx, midx, user_table, movie_table)
    return _mlp_call(
        emb,
        W1, b1.reshape(1, H1),
        W2, b2.reshape(1, H2),
        W3[:, 0].reshape(1, H2), b3.reshape(1, 1),
    )
